# baseline (speedup 1.0000x reference)
import math

import jax
import jax.numpy as jnp
from jax import lax
from jax.experimental import pallas as pl
from jax.experimental.pallas import tpu as pltpu

_LOG01 = math.log(0.1)
_NEG = float("-inf")

_B = 64
_V = 1000000
_CH = 49920
_NB = 20
_TAIL = _V - _NB * _CH


def _insert4(t0, t1, t2, t3, x):
  m0 = jnp.maximum(t0, x)
  w = jnp.minimum(t0, x)
  m1 = jnp.maximum(t1, w)
  w = jnp.minimum(t1, w)
  m2 = jnp.maximum(t2, w)
  w = jnp.minimum(t2, w)
  m3 = jnp.maximum(t3, w)
  return m0, m1, m2, m3


def _body(x_ref, out_ref, t0_ref, t1_ref, t2_ref, t3_ref, s_ref, head_ref):
  b = pl.program_id(0)

  @pl.when(b < _NB)
  def _stream_step():
    first = b == 0
    x0 = x_ref[:, 0:128]
    ninf = jnp.full_like(x0, _NEG)
    t0 = jnp.where(first, ninf, t0_ref[...])
    t1 = jnp.where(first, ninf, t1_ref[...])
    t2 = jnp.where(first, ninf, t2_ref[...])
    t3 = jnp.where(first, ninf, t3_ref[...])
    s = jnp.where(first, 0.0, s_ref[...])
    bref = jnp.where(first, x0, t0)
    acc = jnp.zeros_like(x0)
    for i in range(_CH // 128):
      xs = x_ref[:, 128 * i:128 * (i + 1)]
      t0, t1, t2, t3 = _insert4(t0, t1, t2, t3, xs)
      acc = acc + jnp.exp(xs - bref)
    s = (s + acc) * jnp.exp(bref - t0)
    t0_ref[...] = t0
    t1_ref[...] = t1
    t2_ref[...] = t2
    t3_ref[...] = t3
    s_ref[...] = s

    @pl.when(first)
    def _():
      head_ref[...] = x0

  @pl.when(b == _NB)
  def _final_step():
    t0 = t0_ref[...]
    t1 = t1_ref[...]
    t2 = t2_ref[...]
    t3 = t3_ref[...]
    tref = t0

    acc = jnp.zeros_like(t0)
    nslice = _TAIL // 128
    for i in range(nslice):
      xs = x_ref[:, 128 * i:128 * (i + 1)]
      t0, t1, t2, t3 = _insert4(t0, t1, t2, t3, xs)
      acc = acc + jnp.exp(xs - tref)
    rem = _TAIL - nslice * 128
    if rem:
      xr = x_ref[:, nslice * 128:_TAIL]
      xs = jnp.concatenate(
          [xr, jnp.full((_B, 128 - rem), _NEG, jnp.float32)], axis=1)
      t0, t1, t2, t3 = _insert4(t0, t1, t2, t3, xs)
      acc = acc + jnp.exp(xs - tref)

    mrow = jnp.max(t0, axis=1, keepdims=True)
    stot = jnp.sum((s_ref[...] + acc) * jnp.exp(tref - mrow), axis=1,
                   keepdims=True)
    lse = mrow + jnp.log(stot)

    cat = jnp.concatenate([t0, t1, t2, t3], axis=1)
    idx = lax.broadcasted_iota(jnp.int32, cat.shape, 1)
    big = jnp.int32(1 << 30)
    gs = []
    for _ in range(4):
      gk = jnp.max(cat, axis=1, keepdims=True)
      eq = cat == gk
      fidx = jnp.min(jnp.where(eq, idx, big), axis=1, keepdims=True)
      cat = jnp.where(idx == fidx, _NEG, cat)
      gs.append(gk)

    thr = mrow + _LOG01
    cnt = ((gs[0] >= thr).astype(jnp.int32)
           + (gs[1] >= thr).astype(jnp.int32)
           + (gs[2] >= thr).astype(jnp.int32)
           + (gs[3] >= thr).astype(jnp.int32))

    head = head_ref[...]
    hidx = lax.broadcasted_iota(jnp.int32, head.shape, 1)
    fm = head < thr
    fbs = []
    for _ in range(3):
      fidx = jnp.min(jnp.where(fm, hidx, big), axis=1, keepdims=True)
      fbs.append(jnp.sum(jnp.where(hidx == fidx, head, 0.0), axis=1,
                         keepdims=True))
      fm = fm & (hidx != fidx)

    chosen = []
    for k in range(4):
      fb = fbs[2]
      if k >= 1:
        fb = jnp.where(cnt == k - 1, fbs[1], fb)
      fb = jnp.where(cnt == k, fbs[0], fb)
      chosen.append(jnp.where(cnt > k, gs[k], fb))

    c0, c1, c2, c3 = chosen
    a, b2 = jnp.maximum(c0, c1), jnp.minimum(c0, c1)
    c, d = jnp.maximum(c2, c3), jnp.minimum(c2, c3)
    c0, c2 = jnp.maximum(a, c), jnp.minimum(a, c)
    c1, c3 = jnp.maximum(b2, d), jnp.minimum(b2, d)
    c1, c2 = jnp.maximum(c1, c2), jnp.minimum(c1, c2)

    col = lax.broadcasted_iota(jnp.int32, (_B, 128), 1)
    outv = jnp.where(col == 0, c0, 0.0)
    outv = jnp.where(col == 1, c1, outv)
    outv = jnp.where(col == 2, c2, outv)
    outv = jnp.where(col == 3, c3, outv)
    out_ref[...] = jnp.where(col < 4, outv - lse, 0.0)


def _run(x):
  scratch = [pltpu.VMEM((_B, 128), jnp.float32)] * 6
  return pl.pallas_call(
      _body,
      grid=(_NB + 1,),
      in_specs=[pl.BlockSpec((_B, _CH), lambda b: (0, b))],
      out_specs=pl.BlockSpec((_B, 128), lambda b: (0, 0)),
      out_shape=jax.ShapeDtypeStruct((_B, 128), jnp.float32),
      scratch_shapes=scratch,
  )(x)


@jax.jit
def kernel(logits, scores, beams):
  del beams
  out = _run(logits)
  return out[:, :4].reshape(-1) + jnp.repeat(scores, 4)


# speedup vs baseline: 1.0410x; 1.0410x over previous
import math

import jax
import jax.numpy as jnp
from jax import lax
from jax.experimental import pallas as pl

_LOG01 = math.log(0.1)
_NEG = float("-inf")

_B = 64
_V = 1000000
_CH = 49920
_NB = 20
_TAIL = _V - _NB * _CH


def _insert4(t0, t1, t2, t3, x):
  m0 = jnp.maximum(t0, x)
  w = jnp.minimum(t0, x)
  m1 = jnp.maximum(t1, w)
  w = jnp.minimum(t1, w)
  m2 = jnp.maximum(t2, w)
  w = jnp.minimum(t2, w)
  m3 = jnp.maximum(t3, w)
  return m0, m1, m2, m3


def _stream_body(x_ref, t0_ref, t1_ref, t2_ref, t3_ref, s_ref):
  first = pl.program_id(0) == 0
  x0 = x_ref[:, 0:128]
  ninf = jnp.full_like(x0, _NEG)
  t0 = jnp.where(first, ninf, t0_ref[...])
  t1 = jnp.where(first, ninf, t1_ref[...])
  t2 = jnp.where(first, ninf, t2_ref[...])
  t3 = jnp.where(first, ninf, t3_ref[...])
  s = jnp.where(first, 0.0, s_ref[...])
  bref = jnp.where(first, x0, t0)
  acc = jnp.zeros_like(x0)
  for i in range(_CH // 128):
    xs = x_ref[:, 128 * i:128 * (i + 1)]
    t0, t1, t2, t3 = _insert4(t0, t1, t2, t3, xs)
    acc = acc + jnp.exp(xs - bref)
  s = (s + acc) * jnp.exp(bref - t0)
  t0_ref[...] = t0
  t1_ref[...] = t1
  t2_ref[...] = t2
  t3_ref[...] = t3
  s_ref[...] = s


def _stream(x):
  part = jax.ShapeDtypeStruct((_B, 128), jnp.float32)
  return pl.pallas_call(
      _stream_body,
      grid=(_NB,),
      in_specs=[pl.BlockSpec((_B, _CH), lambda b: (0, b))],
      out_specs=[pl.BlockSpec((_B, 128), lambda b: (0, 0))] * 5,
      out_shape=[part] * 5,
  )(x)


def _finalize_body(t0_ref, t1_ref, t2_ref, t3_ref, s_ref, head_ref, tail_ref,
                   out_ref):
  t0 = t0_ref[...]
  t1 = t1_ref[...]
  t2 = t2_ref[...]
  t3 = t3_ref[...]
  tref = t0

  acc = jnp.zeros_like(t0)
  nslice = _TAIL // 128
  for i in range(nslice):
    xs = tail_ref[:, 128 * i:128 * (i + 1)]
    t0, t1, t2, t3 = _insert4(t0, t1, t2, t3, xs)
    acc = acc + jnp.exp(xs - tref)
  rem = _TAIL - nslice * 128
  if rem:
    xr = tail_ref[:, nslice * 128:_TAIL]
    xs = jnp.concatenate(
        [xr, jnp.full((_B, 128 - rem), _NEG, jnp.float32)], axis=1)
    t0, t1, t2, t3 = _insert4(t0, t1, t2, t3, xs)
    acc = acc + jnp.exp(xs - tref)

  mrow = jnp.max(t0, axis=1, keepdims=True)
  stot = jnp.sum((s_ref[...] + acc) * jnp.exp(tref - mrow), axis=1,
                 keepdims=True)
  lse = mrow + jnp.log(stot)

  cat = jnp.concatenate([t0, t1, t2, t3], axis=1)
  idx = lax.broadcasted_iota(jnp.int32, cat.shape, 1)
  big = jnp.int32(1 << 30)
  gs = []
  for _ in range(4):
    gk = jnp.max(cat, axis=1, keepdims=True)
    eq = cat == gk
    fidx = jnp.min(jnp.where(eq, idx, big), axis=1, keepdims=True)
    cat = jnp.where(idx == fidx, _NEG, cat)
    gs.append(gk)

  thr = mrow + _LOG01
  cnt = ((gs[0] >= thr).astype(jnp.int32) + (gs[1] >= thr).astype(jnp.int32)
         + (gs[2] >= thr).astype(jnp.int32)
         + (gs[3] >= thr).astype(jnp.int32))

  head = head_ref[...]
  hidx = lax.broadcasted_iota(jnp.int32, head.shape, 1)
  fm = head < thr
  fbs = []
  for _ in range(3):
    fidx = jnp.min(jnp.where(fm, hidx, big), axis=1, keepdims=True)
    fbs.append(jnp.sum(jnp.where(hidx == fidx, head, 0.0), axis=1,
                       keepdims=True))
    fm = fm & (hidx != fidx)

  chosen = []
  for k in range(4):
    fb = fbs[2]
    if k >= 1:
      fb = jnp.where(cnt == k - 1, fbs[1], fb)
    fb = jnp.where(cnt == k, fbs[0], fb)
    chosen.append(jnp.where(cnt > k, gs[k], fb))

  c0, c1, c2, c3 = chosen
  a, b = jnp.maximum(c0, c1), jnp.minimum(c0, c1)
  c, d = jnp.maximum(c2, c3), jnp.minimum(c2, c3)
  c0, c2 = jnp.maximum(a, c), jnp.minimum(a, c)
  c1, c3 = jnp.maximum(b, d), jnp.minimum(b, d)
  c1, c2 = jnp.maximum(c1, c2), jnp.minimum(c1, c2)

  col = lax.broadcasted_iota(jnp.int32, (_B, 128), 1)
  outv = jnp.where(col == 0, c0, 0.0)
  outv = jnp.where(col == 1, c1, outv)
  outv = jnp.where(col == 2, c2, outv)
  outv = jnp.where(col == 3, c3, outv)
  out_ref[...] = jnp.where(col < 4, outv - lse, 0.0)


def _finalize(parts, x):
  full = pl.BlockSpec((_B, 128), lambda i: (0, 0))
  tail = pl.BlockSpec((_B, 6400), lambda i: (0, (_NB * _CH) // 6400))
  return pl.pallas_call(
      _finalize_body,
      grid=(1,),
      in_specs=[full] * 5 + [full, tail],
      out_specs=full,
      out_shape=jax.ShapeDtypeStruct((_B, 128), jnp.float32),
  )(*parts, x, x)


@jax.jit
def kernel(logits, scores, beams):
  del beams
  parts = _stream(logits)
  out = _finalize(parts, logits)
  return out[:, :4].reshape(-1) + jnp.repeat(scores, 4)
